# baseline (speedup 1.0000x reference)
import jax
import jax.numpy as jnp
from jax import lax
from jax.experimental import pallas as pl
from jax.experimental.pallas import tpu as pltpu
from jax.experimental.pallas import tpu_sc as plsc

D_MODEL = 1024
SEQ_LEN = 2048
BATCH = 4
NUM_CORES = 2
NUM_SUBCORES = 16
NUM_WORKERS = NUM_CORES * NUM_SUBCORES
POS_PER_WORKER = SEQ_LEN // NUM_WORKERS
CHUNK_POS = 4
NUM_CHUNKS = POS_PER_WORKER // CHUNK_POS
ROWS_PER_CHUNK = BATCH * CHUNK_POS
NBUF = 3
LANES = 16


def _body(ids_hbm, wtab_hbm, ptab_hbm, out_hbm, idx_v, wbuf, pbuf,
          wsem0, wsem1, wsem2, psem0, psem1, psem2, osem0, osem1, osem2):
    wsem = (wsem0, wsem1, wsem2)
    psem = (psem0, psem1, psem2)
    osem = (osem0, osem1, osem2)
    wid = lax.axis_index("s") * NUM_CORES + lax.axis_index("c")
    p0 = wid * POS_PER_WORKER
    i0 = wid * BATCH * POS_PER_WORKER

    gathers = [None] * NUM_CHUNKS
    poscopies = [None] * NUM_CHUNKS
    stores = [None] * NUM_CHUNKS

    def issue(t):
        par = t % NBUF
        gathers[t] = pltpu.async_copy(
            wtab_hbm.at[idx_v.at[pl.ds(t * ROWS_PER_CHUNK, ROWS_PER_CHUNK)]],
            wbuf.at[par], wsem[par])
        poscopies[t] = pltpu.async_copy(
            ptab_hbm.at[pl.ds(p0 + t * CHUNK_POS, CHUNK_POS)],
            pbuf.at[par], psem[par])

    pltpu.sync_copy(ids_hbm.at[pl.ds(i0, ROWS_PER_CHUNK)],
                    idx_v.at[pl.ds(0, ROWS_PER_CHUNK)])
    issue(0)
    pltpu.sync_copy(
        ids_hbm.at[pl.ds(i0 + ROWS_PER_CHUNK,
                         (NUM_CHUNKS - 1) * ROWS_PER_CHUNK)],
        idx_v.at[pl.ds(ROWS_PER_CHUNK, (NUM_CHUNKS - 1) * ROWS_PER_CHUNK)])
    issue(1)

    for t in range(NUM_CHUNKS):
        par = t % NBUF
        gathers[t].wait()
        poscopies[t].wait()

        def add_pos(p, carry):
            @plsc.parallel_loop(0, D_MODEL, step=LANES, unroll=8)
            def add_vec(o):
                js = pl.ds(o, LANES)
                pv = pbuf[par, p, js]
                for b in range(BATCH):
                    wbuf[par, b * CHUNK_POS + p, js] = (
                        wbuf[par, b * CHUNK_POS + p, js] + pv)

            return carry

        lax.fori_loop(0, CHUNK_POS, add_pos, 0)

        stores[t] = [
            pltpu.async_copy(
                wbuf.at[par, pl.ds(b * CHUNK_POS, CHUNK_POS)],
                out_hbm.at[b, pl.ds(p0 + t * CHUNK_POS, CHUNK_POS), :],
                osem[par])
            for b in range(BATCH)
        ]

        if t + 2 < NUM_CHUNKS:
            if t >= 1:
                for c in stores[t - 1]:
                    c.wait()
            issue(t + 2)

    for t in (NUM_CHUNKS - 3, NUM_CHUNKS - 2, NUM_CHUNKS - 1):
        for c in stores[t]:
            c.wait()


_embed = pl.kernel(
    _body,
    out_type=jax.ShapeDtypeStruct((BATCH, SEQ_LEN, D_MODEL), jnp.float32),
    mesh=plsc.VectorSubcoreMesh(core_axis_name="c", subcore_axis_name="s"),
    scratch_types=[
        pltpu.VMEM((BATCH * POS_PER_WORKER,), jnp.int32),
        pltpu.VMEM((NBUF, ROWS_PER_CHUNK, D_MODEL), jnp.float32),
        pltpu.VMEM((NBUF, CHUNK_POS, D_MODEL), jnp.float32),
        pltpu.SemaphoreType.DMA,
        pltpu.SemaphoreType.DMA,
        pltpu.SemaphoreType.DMA,
        pltpu.SemaphoreType.DMA,
        pltpu.SemaphoreType.DMA,
        pltpu.SemaphoreType.DMA,
        pltpu.SemaphoreType.DMA,
        pltpu.SemaphoreType.DMA,
        pltpu.SemaphoreType.DMA,
    ],
)


@jax.jit
def kernel(input_ids, word_embedding, position_embedding):
    batch, seq = input_ids.shape
    ids = (input_ids.astype(jnp.int32)
           .reshape(BATCH, NUM_WORKERS, NUM_CHUNKS, CHUNK_POS)
           .transpose(1, 2, 0, 3)
           .reshape(-1))
    return _embed(ids, word_embedding, position_embedding)


# speedup vs baseline: 1.0765x; 1.0765x over previous
import jax
import jax.numpy as jnp
from jax import lax
from jax.experimental import pallas as pl
from jax.experimental.pallas import tpu as pltpu
from jax.experimental.pallas import tpu_sc as plsc

D_MODEL = 1024
SEQ_LEN = 2048
BATCH = 4
NUM_CORES = 2
NUM_SUBCORES = 16
NUM_WORKERS = NUM_CORES * NUM_SUBCORES
POS_PER_WORKER = SEQ_LEN // NUM_WORKERS
CHUNK_POS = 8
NUM_CHUNKS = POS_PER_WORKER // CHUNK_POS
ROWS_PER_CHUNK = BATCH * CHUNK_POS
NBUF = 3
LANES = 16


def _body(ids_hbm, wtab_hbm, ptab_hbm, out_hbm, idx_v, wbuf, pbuf,
          wsem0, wsem1, wsem2, psem0, psem1, psem2, osem0, osem1, osem2):
    wsem = (wsem0, wsem1, wsem2)
    psem = (psem0, psem1, psem2)
    osem = (osem0, osem1, osem2)
    wid = lax.axis_index("s") * NUM_CORES + lax.axis_index("c")
    p0 = wid * POS_PER_WORKER
    i0 = wid * BATCH * POS_PER_WORKER

    gathers = [None] * NUM_CHUNKS
    poscopies = [None] * NUM_CHUNKS
    stores = [None] * NUM_CHUNKS

    def issue(t):
        par = t % NBUF
        gathers[t] = pltpu.async_copy(
            wtab_hbm.at[idx_v.at[pl.ds(t * ROWS_PER_CHUNK, ROWS_PER_CHUNK)]],
            wbuf.at[par], wsem[par])
        poscopies[t] = pltpu.async_copy(
            ptab_hbm.at[pl.ds(p0 + t * CHUNK_POS, CHUNK_POS)],
            pbuf.at[par], psem[par])

    pltpu.sync_copy(ids_hbm.at[pl.ds(i0, ROWS_PER_CHUNK)],
                    idx_v.at[pl.ds(0, ROWS_PER_CHUNK)])
    issue(0)
    pltpu.sync_copy(
        ids_hbm.at[pl.ds(i0 + ROWS_PER_CHUNK,
                         (NUM_CHUNKS - 1) * ROWS_PER_CHUNK)],
        idx_v.at[pl.ds(ROWS_PER_CHUNK, (NUM_CHUNKS - 1) * ROWS_PER_CHUNK)])
    issue(1)

    for t in range(NUM_CHUNKS):
        par = t % NBUF
        gathers[t].wait()
        poscopies[t].wait()

        def add_pos(p, carry):
            @plsc.parallel_loop(0, D_MODEL, step=LANES, unroll=8)
            def add_vec(o):
                js = pl.ds(o, LANES)
                pv = pbuf[par, p, js]
                for b in range(BATCH):
                    wbuf[par, b * CHUNK_POS + p, js] = (
                        wbuf[par, b * CHUNK_POS + p, js] + pv)

            return carry

        lax.fori_loop(0, CHUNK_POS, add_pos, 0)

        stores[t] = [
            pltpu.async_copy(
                wbuf.at[par, pl.ds(b * CHUNK_POS, CHUNK_POS)],
                out_hbm.at[b, pl.ds(p0 + t * CHUNK_POS, CHUNK_POS), :],
                osem[par])
            for b in range(BATCH)
        ]

        if t + 2 < NUM_CHUNKS:
            if t >= 1:
                for c in stores[t - 1]:
                    c.wait()
            issue(t + 2)

    for t in (NUM_CHUNKS - 3, NUM_CHUNKS - 2, NUM_CHUNKS - 1):
        for c in stores[t]:
            c.wait()


_embed = pl.kernel(
    _body,
    out_type=jax.ShapeDtypeStruct((BATCH, SEQ_LEN, D_MODEL), jnp.float32),
    mesh=plsc.VectorSubcoreMesh(core_axis_name="c", subcore_axis_name="s"),
    scratch_types=[
        pltpu.VMEM((BATCH * POS_PER_WORKER,), jnp.int32),
        pltpu.VMEM((NBUF, ROWS_PER_CHUNK, D_MODEL), jnp.float32),
        pltpu.VMEM((NBUF, CHUNK_POS, D_MODEL), jnp.float32),
        pltpu.SemaphoreType.DMA,
        pltpu.SemaphoreType.DMA,
        pltpu.SemaphoreType.DMA,
        pltpu.SemaphoreType.DMA,
        pltpu.SemaphoreType.DMA,
        pltpu.SemaphoreType.DMA,
        pltpu.SemaphoreType.DMA,
        pltpu.SemaphoreType.DMA,
        pltpu.SemaphoreType.DMA,
    ],
)


@jax.jit
def kernel(input_ids, word_embedding, position_embedding):
    batch, seq = input_ids.shape
    ids = (input_ids.astype(jnp.int32)
           .reshape(BATCH, NUM_WORKERS, NUM_CHUNKS, CHUNK_POS)
           .transpose(1, 2, 0, 3)
           .reshape(-1))
    return _embed(ids, word_embedding, position_embedding)
